# baseline (speedup 1.0000x reference)
import jax
import jax.numpy as jnp
from jax import lax
from jax.experimental import pallas as pl
from jax.experimental.pallas import tpu as pltpu

_N = 10000
_BLOCK = 2048
_STEPS = 5
_PAD = _STEPS * _BLOCK

_DN_RT = (((1,), (1,)), ((), ()))


def _cell_body(x_ref, ht_ref, wzt_ref, wrt_ref, wht_ref, b_ref, wlt_ref,
               blt_ref, outt_ref, hnewt_ref, hnat_scr, hnew_scr, out_scr):
    i = pl.program_id(0)
    d_in = x_ref.shape[1]

    @pl.when(i == 0)
    def _load_h():
        hnat_scr[:_N] = jnp.transpose(ht_ref[...])

    wz = wzt_ref[0, 0] + wzt_ref[1, 0]
    wr = wrt_ref[0, 0] + wrt_ref[1, 0]
    wh = wht_ref[0, 0] + wht_ref[1, 0]
    w_all = jnp.concatenate([wz, wr, wh], axis=0)
    rows = pl.ds(i * _BLOCK, _BLOCK)
    x_b = x_ref[...]
    h_nat = hnat_scr[rows]
    gx = (lax.dot_general(x_b, w_all[:, :d_in], _DN_RT,
                          preferred_element_type=jnp.float32)
          + b_ref[...])
    zr = jax.nn.sigmoid(
        gx[:, :64]
        + lax.dot_general(h_nat, w_all[:64, d_in:], _DN_RT,
                          preferred_element_type=jnp.float32))
    z = zr[:, :32]
    r = zr[:, 32:]
    htl = jnp.tanh(
        gx[:, 64:]
        + lax.dot_general(h_nat * r, wh[:, d_in:], _DN_RT,
                          preferred_element_type=jnp.float32))
    h_new = z * h_nat + (1.0 - z) * htl
    hnew_scr[rows] = h_new
    out_scr[rows] = (lax.dot_general(jnp.maximum(h_new, 0.0), wlt_ref[...],
                                     _DN_RT, preferred_element_type=jnp.float32)
                     + blt_ref[...])

    @pl.when(i == _STEPS - 1)
    def _store():
        outt_ref[...] = jnp.transpose(out_scr[:_N])
        hnewt_ref[...] = jnp.transpose(hnew_scr[:_N])


def kernel(x, edge_index, edge_weight, h, Wz, bz, Wr, br, Wh, bh, W_lin, b_lin):
    del edge_index, edge_weight
    d_hid = h.shape[1]
    d_out = W_lin.shape[1]
    ht = h.T
    wzt = jnp.transpose(Wz, (0, 1, 3, 2))
    wrt = jnp.transpose(Wr, (0, 1, 3, 2))
    wht = jnp.transpose(Wh, (0, 1, 3, 2))
    wlt = W_lin.T
    b_all = jnp.concatenate([bz, br, bh])[None]
    blt = b_lin[None]

    full = lambda a: pl.BlockSpec(a.shape, lambda i: (0,) * a.ndim)
    out_t, h_new_t = pl.pallas_call(
        _cell_body,
        grid=(_STEPS,),
        in_specs=[
            pl.BlockSpec((_BLOCK, x.shape[1]), lambda i: (i, 0)),
            full(ht), full(wzt), full(wrt), full(wht),
            full(b_all), full(wlt), full(blt),
        ],
        out_specs=[
            pl.BlockSpec((d_out, _N), lambda i: (0, 0)),
            pl.BlockSpec((d_hid, _N), lambda i: (0, 0)),
        ],
        out_shape=[
            jax.ShapeDtypeStruct((d_out, _N), jnp.float32),
            jax.ShapeDtypeStruct((d_hid, _N), jnp.float32),
        ],
        scratch_shapes=[
            pltpu.VMEM((_PAD, d_hid), jnp.float32),
            pltpu.VMEM((_PAD, d_hid), jnp.float32),
            pltpu.VMEM((_PAD, d_out), jnp.float32),
        ],
        compiler_params=pltpu.CompilerParams(
            dimension_semantics=("arbitrary",),
        ),
    )(x, ht, wzt, wrt, wht, b_all, wlt, blt)
    return (out_t.T, h_new_t.T)


# speedup vs baseline: 1.1898x; 1.1898x over previous
import jax
import jax.numpy as jnp
from jax import lax
from jax.experimental import pallas as pl
from jax.experimental.pallas import tpu as pltpu

_N = 10000
_BLOCK = 2048
_STEPS = 5

_DN_RT = (((1,), (1,)), ((), ()))


def _cell_body(x_ref, ht_ref, wzt_ref, wrt_ref, wht_ref, b_ref, wlt_ref,
               blt_ref, outt_ref, hnewt_ref):
    d_in = x_ref.shape[1]
    wz = wzt_ref[0, 0] + wzt_ref[1, 0]
    wr = wrt_ref[0, 0] + wrt_ref[1, 0]
    wh = wht_ref[0, 0] + wht_ref[1, 0]
    w_all = jnp.concatenate([wz, wr, wh], axis=0)
    x_b = x_ref[...]
    h_nat = jnp.transpose(ht_ref[...])
    gx = (lax.dot_general(x_b, w_all[:, :d_in], _DN_RT,
                          preferred_element_type=jnp.float32)
          + b_ref[...])
    zr = jax.nn.sigmoid(
        gx[:, :64]
        + lax.dot_general(h_nat, w_all[:64, d_in:], _DN_RT,
                          preferred_element_type=jnp.float32))
    z = zr[:, :32]
    r = zr[:, 32:]
    htl = jnp.tanh(
        gx[:, 64:]
        + lax.dot_general(h_nat * r, wh[:, d_in:], _DN_RT,
                          preferred_element_type=jnp.float32))
    h_new = z * h_nat + (1.0 - z) * htl
    hnewt_ref[...] = jnp.transpose(h_new)
    out = (lax.dot_general(jnp.maximum(h_new, 0.0), wlt_ref[...], _DN_RT,
                           preferred_element_type=jnp.float32)
           + blt_ref[...])
    outt_ref[...] = jnp.transpose(out)


def kernel(x, edge_index, edge_weight, h, Wz, bz, Wr, br, Wh, bh, W_lin, b_lin):
    del edge_index, edge_weight
    d_hid = h.shape[1]
    d_out = W_lin.shape[1]
    ht = h.T
    wzt = jnp.transpose(Wz, (0, 1, 3, 2))
    wrt = jnp.transpose(Wr, (0, 1, 3, 2))
    wht = jnp.transpose(Wh, (0, 1, 3, 2))
    wlt = W_lin.T
    b_all = jnp.concatenate([bz, br, bh])[None]
    blt = b_lin[None]

    full = lambda a: pl.BlockSpec(a.shape, lambda i: (0,) * a.ndim)
    out_t, h_new_t = pl.pallas_call(
        _cell_body,
        grid=(_STEPS,),
        in_specs=[
            pl.BlockSpec((_BLOCK, x.shape[1]), lambda i: (i, 0)),
            pl.BlockSpec((d_hid, _BLOCK), lambda i: (0, i)),
            full(wzt), full(wrt), full(wht),
            full(b_all), full(wlt), full(blt),
        ],
        out_specs=[
            pl.BlockSpec((d_out, _BLOCK), lambda i: (0, i)),
            pl.BlockSpec((d_hid, _BLOCK), lambda i: (0, i)),
        ],
        out_shape=[
            jax.ShapeDtypeStruct((d_out, _N), jnp.float32),
            jax.ShapeDtypeStruct((d_hid, _N), jnp.float32),
        ],
        compiler_params=pltpu.CompilerParams(
            dimension_semantics=("parallel",),
        ),
    )(x, ht, wzt, wrt, wht, b_all, wlt, blt)
    return (out_t.T, h_new_t.T)


# speedup vs baseline: 1.2168x; 1.0227x over previous
import jax
import jax.numpy as jnp
from jax import lax
from jax.experimental import pallas as pl
from jax.experimental.pallas import tpu as pltpu

_N = 10000

_DN_RT = (((1,), (1,)), ((), ()))


def _cell_body(x_ref, ht_ref, wzt_ref, wrt_ref, wht_ref, b_ref, wlt_ref,
               blt_ref, outt_ref, hnewt_ref):
    d_in = x_ref.shape[1]
    wz = wzt_ref[0, 0] + wzt_ref[1, 0]
    wr = wrt_ref[0, 0] + wrt_ref[1, 0]
    wh = wht_ref[0, 0] + wht_ref[1, 0]
    w_all = jnp.concatenate([wz, wr, wh], axis=0)
    x_b = x_ref[...]
    ht_b = ht_ref[...]
    h_nat = jnp.transpose(ht_b)
    gx = (lax.dot_general(x_b, w_all[:, :d_in], _DN_RT,
                          preferred_element_type=jnp.float32)
          + b_ref[...])
    zr = jax.nn.sigmoid(
        gx[:, :64]
        + lax.dot_general(h_nat, w_all[:64, d_in:], _DN_RT,
                          preferred_element_type=jnp.float32))
    z = zr[:, :32]
    r = zr[:, 32:]
    htl = jnp.tanh(
        gx[:, 64:]
        + lax.dot_general(h_nat * r, wh[:, d_in:], _DN_RT,
                          preferred_element_type=jnp.float32))
    h_new = z * h_nat + (1.0 - z) * htl
    h_new_t = jnp.transpose(h_new)
    hnewt_ref[...] = h_new_t
    outt_ref[...] = (jnp.dot(wlt_ref[...], jnp.maximum(h_new_t, 0.0),
                             preferred_element_type=jnp.float32)
                     + jnp.transpose(blt_ref[...]))


def kernel(x, edge_index, edge_weight, h, Wz, bz, Wr, br, Wh, bh, W_lin, b_lin):
    del edge_index, edge_weight
    d_hid = h.shape[1]
    d_out = W_lin.shape[1]
    ht = h.T
    wzt = jnp.transpose(Wz, (0, 1, 3, 2))
    wrt = jnp.transpose(Wr, (0, 1, 3, 2))
    wht = jnp.transpose(Wh, (0, 1, 3, 2))
    wlt = W_lin.T
    b_all = jnp.concatenate([bz, br, bh])[None]
    blt = b_lin[None]

    full = lambda a: pl.BlockSpec(a.shape, lambda: (0,) * a.ndim)
    out_t, h_new_t = pl.pallas_call(
        _cell_body,
        grid=(),
        in_specs=[full(x), full(ht), full(wzt), full(wrt), full(wht),
                  full(b_all), full(wlt), full(blt)],
        out_specs=[
            pl.BlockSpec((d_out, _N), lambda: (0, 0)),
            pl.BlockSpec((d_hid, _N), lambda: (0, 0)),
        ],
        out_shape=[
            jax.ShapeDtypeStruct((d_out, _N), jnp.float32),
            jax.ShapeDtypeStruct((d_hid, _N), jnp.float32),
        ],
    )(x, ht, wzt, wrt, wht, b_all, wlt, blt)
    return (out_t.T, h_new_t.T)


# speedup vs baseline: 1.9036x; 1.5644x over previous
import jax
import jax.numpy as jnp
from jax import lax
from jax.experimental import pallas as pl
from jax.experimental.pallas import tpu as pltpu

_N = 10000

_DN_RT = (((1,), (1,)), ((), ()))


def _cell_body(x_ref, ht_ref, wzt_ref, wrt_ref, wht_ref, b_ref, wlt_ref,
               blt_ref, outt_ref, hnewt_ref):
    d_in = x_ref.shape[1]
    wz = wzt_ref[0, 0] + wzt_ref[1, 0]
    wr = wrt_ref[0, 0] + wrt_ref[1, 0]
    wh = wht_ref[0, 0] + wht_ref[1, 0]
    w_all = jnp.concatenate([wz, wr, wh], axis=0)
    x_b = x_ref[...]
    ht_b = ht_ref[...]
    gx = lax.dot_general(x_b, w_all[:, :d_in], _DN_RT,
                         preferred_element_type=jnp.float32)
    gxt = jnp.transpose(gx) + jnp.transpose(b_ref[...])
    zr = jax.nn.sigmoid(
        gxt[:64]
        + jnp.dot(w_all[:64, d_in:], ht_b,
                  preferred_element_type=jnp.float32))
    z = zr[:32]
    r = zr[32:]
    htl = jnp.tanh(
        gxt[64:]
        + jnp.dot(wh[:, d_in:], r * ht_b,
                  preferred_element_type=jnp.float32))
    h_new = z * ht_b + (1.0 - z) * htl
    hnewt_ref[...] = h_new
    outt_ref[...] = (jnp.dot(wlt_ref[...], jnp.maximum(h_new, 0.0),
                             preferred_element_type=jnp.float32)
                     + jnp.transpose(blt_ref[...]))


def kernel(x, edge_index, edge_weight, h, Wz, bz, Wr, br, Wh, bh, W_lin, b_lin):
    del edge_index, edge_weight
    d_hid = h.shape[1]
    d_out = W_lin.shape[1]
    ht = h.T
    wzt = jnp.transpose(Wz, (0, 1, 3, 2))
    wrt = jnp.transpose(Wr, (0, 1, 3, 2))
    wht = jnp.transpose(Wh, (0, 1, 3, 2))
    wlt = W_lin.T
    b_all = jnp.concatenate([bz, br, bh])[None]
    blt = b_lin[None]

    full = lambda a: pl.BlockSpec(a.shape, lambda: (0,) * a.ndim)
    out_t, h_new_t = pl.pallas_call(
        _cell_body,
        grid=(),
        in_specs=[full(x), full(ht), full(wzt), full(wrt), full(wht),
                  full(b_all), full(wlt), full(blt)],
        out_specs=[
            pl.BlockSpec((d_out, _N), lambda: (0, 0)),
            pl.BlockSpec((d_hid, _N), lambda: (0, 0)),
        ],
        out_shape=[
            jax.ShapeDtypeStruct((d_out, _N), jnp.float32),
            jax.ShapeDtypeStruct((d_hid, _N), jnp.float32),
        ],
    )(x, ht, wzt, wrt, wht, b_all, wlt, blt)
    return (out_t.T, h_new_t.T)
